# baseline (speedup 1.0000x reference)
import functools

import jax
import jax.numpy as jnp
from jax import lax
from jax.experimental import pallas as pl
from jax.experimental.pallas import tpu as pltpu
from jax.experimental.pallas import tpu_sc as plsc

VOCAB = 1000000
DIM = 32
BATCH = 16384
SEQ = 50
TOTAL = BATCH * SEQ
NW = 32
BPW = BATCH // NW
NBLK = BPW // 128

_mesh = plsc.VectorSubcoreMesh(core_axis_name="c", subcore_axis_name="s")


@functools.partial(
    pl.kernel,
    mesh=_mesh,
    compiler_params=pltpu.CompilerParams(use_tc_tiling_on_sc=False,
                                         needs_layout_passes=False),
    out_type=jax.ShapeDtypeStruct((SEQ, DIM, BATCH), jnp.float32),
    scratch_types=[
        pltpu.VMEM((BPW,), jnp.int32),
        pltpu.VMEM((BPW,), jnp.int32),
        pltpu.VMEM((BPW, DIM), jnp.float32),
        pltpu.VMEM((BPW, DIM), jnp.float32),
        pltpu.VMEM((DIM * BPW,), jnp.float32),
        pltpu.VMEM((DIM * BPW,), jnp.float32),
        pltpu.SemaphoreType.DMA,
        pltpu.SemaphoreType.DMA,
        pltpu.SemaphoreType.DMA,
        pltpu.SemaphoreType.DMA,
        pltpu.SemaphoreType.DMA,
        pltpu.SemaphoreType.DMA,
    ],
)
def _lookup_kernel(idx_hbm, table_hbm, out_hbm, idx_a, idx_b, rows_a,
                   rows_b, tr_a, tr_b, sia, sib, sga, sgb, swa, swb):
    wid = lax.axis_index("s") * 2 + lax.axis_index("c")
    b0 = wid * BPW

    def idx_load(s, buf, sem):
        return pltpu.async_copy(idx_hbm.at[s, pl.ds(b0, BPW)], buf, sem)

    def gather(ibuf, rbuf, sem):
        return pltpu.async_copy(table_hbm.at[ibuf], rbuf, sem)

    lane = lax.iota(jnp.int32, 16)

    l0 = lane * BPW
    l1 = l0 + 16 * BPW

    def drain_writes(s, tbuf, swx):
        def wbody(d, carry):
            pltpu.make_async_copy(
                tbuf.at[pl.ds(d * BPW, BPW)],
                out_hbm.at[s, d, pl.ds(b0, BPW)], swx).wait()
            return carry
        lax.fori_loop(0, DIM, wbody, 0)

    def transpose_write(t, s, rbuf, tbuf, swx):
        @pl.when(t > 0)
        def _():
            drain_writes(s, tbuf, swx)

        def rbody(r, carry):
            base = r * 16
            for j in range(16):
                bl = base + j
                v0 = rbuf[bl, pl.ds(0, 16)]
                v1 = rbuf[bl, pl.ds(16, 16)]
                plsc.store_scatter(tbuf, [l0 + bl], v0)
                plsc.store_scatter(tbuf, [l1 + bl], v1)
            return carry

        lax.fori_loop(0, BPW // 16, rbody, 0)

        def dbody(d, carry):
            pltpu.async_copy(tbuf.at[pl.ds(d * BPW, BPW)],
                             out_hbm.at[s, d, pl.ds(b0, BPW)], swx)
            return carry
        lax.fori_loop(0, DIM, dbody, 0)

    idx_load(0, idx_a, sia).wait()
    idx_load(1, idx_b, sib)
    gather(idx_a, rows_a, sga)

    def body(t, carry):
        pltpu.make_async_copy(table_hbm.at[idx_a], rows_a, sga).wait()
        pltpu.make_async_copy(
            idx_hbm.at[0, pl.ds(b0, BPW)], idx_b, sib).wait()
        gather(idx_b, rows_b, sgb)

        @pl.when(t < SEQ // 2 - 1)
        def _():
            idx_load(2 * t + 2, idx_a, sia)

        transpose_write(t, 2 * t, rows_a, tr_a, swa)
        pltpu.make_async_copy(table_hbm.at[idx_b], rows_b, sgb).wait()

        @pl.when(t < SEQ // 2 - 1)
        def _():
            pltpu.make_async_copy(
                idx_hbm.at[0, pl.ds(b0, BPW)], idx_a, sia).wait()
            gather(idx_a, rows_a, sga)
            idx_load(2 * t + 3, idx_b, sib)

        transpose_write(t, 2 * t + 1, rows_b, tr_b, swb)
        return carry

    lax.fori_loop(0, SEQ // 2, body, 0)
    drain_writes(0, tr_a, swa)
    drain_writes(0, tr_b, swb)


def kernel(input_data, table):
    out = _lookup_kernel(input_data.T.astype(jnp.int32), table)
    return jnp.transpose(out, (2, 0, 1))


# speedup vs baseline: 1.0097x; 1.0097x over previous
import functools

import jax
import jax.numpy as jnp
from jax import lax
from jax.experimental import pallas as pl
from jax.experimental.pallas import tpu as pltpu
from jax.experimental.pallas import tpu_sc as plsc

VOCAB = 1000000
DIM = 32
BATCH = 16384
SEQ = 50
TOTAL = BATCH * SEQ
NW = 32
BPW = BATCH // NW
NBLK = BPW // 128

_mesh = plsc.VectorSubcoreMesh(core_axis_name="c", subcore_axis_name="s")


@functools.partial(
    pl.kernel,
    mesh=_mesh,
    compiler_params=pltpu.CompilerParams(use_tc_tiling_on_sc=False,
                                         needs_layout_passes=False),
    out_type=jax.ShapeDtypeStruct((SEQ, DIM, BATCH), jnp.float32),
    scratch_types=[
        pltpu.VMEM((BPW,), jnp.int32),
        pltpu.VMEM((BPW,), jnp.int32),
        pltpu.VMEM((BPW, DIM), jnp.float32),
        pltpu.VMEM((BPW, DIM), jnp.float32),
        pltpu.VMEM((DIM, BPW), jnp.float32),
        pltpu.VMEM((DIM, BPW), jnp.float32),
        pltpu.SemaphoreType.DMA,
        pltpu.SemaphoreType.DMA,
        pltpu.SemaphoreType.DMA,
        pltpu.SemaphoreType.DMA,
        pltpu.SemaphoreType.DMA,
        pltpu.SemaphoreType.DMA,
    ],
)
def _lookup_kernel(idx_hbm, table_hbm, out_hbm, idx_a, idx_b, rows_a,
                   rows_b, tr_a, tr_b, sia, sib, sga, sgb, swa, swb):
    wid = lax.axis_index("s") * 2 + lax.axis_index("c")
    b0 = wid * BPW

    def idx_load(s, buf, sem):
        return pltpu.async_copy(idx_hbm.at[s, pl.ds(b0, BPW)], buf, sem)

    def gather(ibuf, rbuf, sem):
        return pltpu.async_copy(table_hbm.at[ibuf], rbuf, sem)

    lane = lax.iota(jnp.int32, 16)

    lane16 = lane + 16

    def drain_writes(s, tbuf, swx):
        pltpu.make_async_copy(
            tbuf, out_hbm.at[s, :, pl.ds(b0, BPW)], swx).wait()

    def transpose_write(t, s, rbuf, tbuf, swx):
        @pl.when(t > 0)
        def _():
            drain_writes(s, tbuf, swx)

        def rbody(r, carry):
            base = r * 16
            for j in range(16):
                bl = base + j
                col = jnp.full((16,), bl, dtype=jnp.int32)
                v0 = rbuf[bl, pl.ds(0, 16)]
                v1 = rbuf[bl, pl.ds(16, 16)]
                plsc.store_scatter(tbuf, [lane, col], v0)
                plsc.store_scatter(tbuf, [lane16, col], v1)
            return carry

        lax.fori_loop(0, BPW // 16, rbody, 0)
        pltpu.async_copy(tbuf, out_hbm.at[s, :, pl.ds(b0, BPW)], swx)

    idx_load(0, idx_a, sia).wait()
    idx_load(1, idx_b, sib)
    gather(idx_a, rows_a, sga)

    def body(t, carry):
        pltpu.make_async_copy(table_hbm.at[idx_a], rows_a, sga).wait()
        pltpu.make_async_copy(
            idx_hbm.at[0, pl.ds(b0, BPW)], idx_b, sib).wait()
        gather(idx_b, rows_b, sgb)

        @pl.when(t < SEQ // 2 - 1)
        def _():
            idx_load(2 * t + 2, idx_a, sia)

        transpose_write(t, 2 * t, rows_a, tr_a, swa)
        pltpu.make_async_copy(table_hbm.at[idx_b], rows_b, sgb).wait()

        @pl.when(t < SEQ // 2 - 1)
        def _():
            pltpu.make_async_copy(
                idx_hbm.at[0, pl.ds(b0, BPW)], idx_a, sia).wait()
            gather(idx_a, rows_a, sga)
            idx_load(2 * t + 3, idx_b, sib)

        transpose_write(t, 2 * t + 1, rows_b, tr_b, swb)
        return carry

    lax.fori_loop(0, SEQ // 2, body, 0)
    drain_writes(0, tr_a, swa)
    drain_writes(0, tr_b, swb)


def kernel(input_data, table):
    out = _lookup_kernel(input_data.T.astype(jnp.int32), table)
    return jnp.transpose(out, (2, 0, 1))


# speedup vs baseline: 1.0752x; 1.0649x over previous
import functools

import jax
import jax.numpy as jnp
from jax import lax
from jax.experimental import pallas as pl
from jax.experimental.pallas import tpu as pltpu
from jax.experimental.pallas import tpu_sc as plsc

VOCAB = 1000000
DIM = 32
TOTAL = 16384 * 50
NW = 32
PER_W = TOTAL // NW
CHUNK = 1600
NCHUNK = PER_W // CHUNK

_mesh = plsc.VectorSubcoreMesh(core_axis_name="c", subcore_axis_name="s")


@functools.partial(
    pl.kernel,
    mesh=_mesh,
    compiler_params=pltpu.CompilerParams(use_tc_tiling_on_sc=False),
    out_type=jax.ShapeDtypeStruct((16384, 50, DIM), jnp.float32),
    scratch_types=[
        pltpu.VMEM((CHUNK,), jnp.int32),
        pltpu.VMEM((CHUNK,), jnp.int32),
        pltpu.VMEM((CHUNK, DIM), jnp.float32),
        pltpu.VMEM((CHUNK, DIM), jnp.float32),
        pltpu.SemaphoreType.DMA,
        pltpu.SemaphoreType.DMA,
        pltpu.SemaphoreType.DMA,
        pltpu.SemaphoreType.DMA,
        pltpu.SemaphoreType.DMA,
        pltpu.SemaphoreType.DMA,
    ],
)
def _gather_kernel(idx_hbm, table_hbm, out_hbm, idx_v0, idx_v1, rows_v0,
                   rows_v1, is0, is1, gs0, gs1, os0, os1):
    wid = lax.axis_index("s") * 2 + lax.axis_index("c")
    base = wid * PER_W

    idx_v = (idx_v0, idx_v1)
    rows_v = (rows_v0, rows_v1)
    isem = (is0, is1)
    gsem = (gs0, gs1)
    osem = (os0, os1)

    def load_idx(i):
        return pltpu.async_copy(
            idx_hbm.at[pl.ds(base + i * CHUNK, CHUNK)], idx_v[i % 2],
            isem[i % 2])

    def gather(i):
        return pltpu.async_copy(table_hbm.at[idx_v[i % 2]], rows_v[i % 2],
                                gsem[i % 2])

    def writeback(i):
        row0 = (base + i * CHUNK) // 50
        return [
            pltpu.async_copy(rows_v[i % 2].at[pl.ds(r * 50, 50)],
                             out_hbm.at[row0 + r], osem[i % 2])
            for r in range(CHUNK // 50)
        ]

    il = [load_idx(0), load_idx(1)]
    il[0].wait()
    g = [gather(0), None]
    ow = [None, None]

    for i in range(NCHUNK):
        b = i % 2
        nb = 1 - b
        g[b].wait()
        if i + 1 < NCHUNK:
            il[nb].wait()
            if ow[nb] is not None:
                for cp in ow[nb]:
                    cp.wait()
            g[nb] = gather(i + 1)
        ow[b] = writeback(i)
        if i + 2 < NCHUNK:
            il[b] = load_idx(i + 2)
    for cp in ow[0]:
        cp.wait()
    for cp in ow[1]:
        cp.wait()


def kernel(input_data, table):
    idx = input_data.reshape(TOTAL).astype(jnp.int32)
    return _gather_kernel(idx, table)


# speedup vs baseline: 1.0774x; 1.0020x over previous
import functools

import jax
import jax.numpy as jnp
from jax import lax
from jax.experimental import pallas as pl
from jax.experimental.pallas import tpu as pltpu
from jax.experimental.pallas import tpu_sc as plsc

VOCAB = 1000000
DIM = 32
TOTAL = 16384 * 50
NW = 32
PER_W = TOTAL // NW
CHUNK = 1600
NCHUNK = PER_W // CHUNK

_mesh = plsc.VectorSubcoreMesh(core_axis_name="c", subcore_axis_name="s")


@functools.partial(
    pl.kernel,
    mesh=_mesh,
    compiler_params=pltpu.CompilerParams(use_tc_tiling_on_sc=False),
    out_type=jax.ShapeDtypeStruct((16384, 50, DIM), jnp.float32),
    scratch_types=[
        pltpu.VMEM((CHUNK // 50, 50), jnp.int32),
        pltpu.VMEM((CHUNK // 50, 50), jnp.int32),
        pltpu.VMEM((CHUNK, DIM), jnp.float32),
        pltpu.VMEM((CHUNK, DIM), jnp.float32),
        pltpu.SemaphoreType.DMA,
        pltpu.SemaphoreType.DMA,
        pltpu.SemaphoreType.DMA,
        pltpu.SemaphoreType.DMA,
        pltpu.SemaphoreType.DMA,
        pltpu.SemaphoreType.DMA,
    ],
)
def _gather_kernel(idx_hbm, table_hbm, out_hbm, idx_v0, idx_v1, rows_v0,
                   rows_v1, is0, is1, gs0, gs1, os0, os1):
    wid = lax.axis_index("s") * 2 + lax.axis_index("c")
    base = wid * PER_W

    idx_v = (idx_v0, idx_v1)
    rows_v = (rows_v0, rows_v1)
    isem = (is0, is1)
    gsem = (gs0, gs1)
    osem = (os0, os1)

    def load_idx(i):
        row0 = (base + i * CHUNK) // 50
        return pltpu.async_copy(
            idx_hbm.at[pl.ds(row0, CHUNK // 50), :], idx_v[i % 2],
            isem[i % 2])

    def gather(i):
        return [
            pltpu.async_copy(table_hbm.at[idx_v[i % 2].at[r]],
                             rows_v[i % 2].at[pl.ds(r * 50, 50)],
                             gsem[i % 2])
            for r in range(CHUNK // 50)
        ]

    def writeback(i):
        row0 = (base + i * CHUNK) // 50
        return [
            pltpu.async_copy(rows_v[i % 2].at[pl.ds(r * 50, 50)],
                             out_hbm.at[row0 + r], osem[i % 2])
            for r in range(CHUNK // 50)
        ]

    il = [load_idx(0), load_idx(1)]
    il[0].wait()
    g = [gather(0), None]
    ow = [None, None]

    for i in range(NCHUNK):
        b = i % 2
        nb = 1 - b
        for cp in g[b]:
            cp.wait()
        if i + 1 < NCHUNK:
            il[nb].wait()
            if ow[nb] is not None:
                for cp in ow[nb]:
                    cp.wait()
            g[nb] = gather(i + 1)
        ow[b] = writeback(i)
        if i + 2 < NCHUNK:
            il[b] = load_idx(i + 2)
    for cp in ow[0]:
        cp.wait()
    for cp in ow[1]:
        cp.wait()


def kernel(input_data, table):
    return _gather_kernel(input_data.astype(jnp.int32), table)
